# Initial kernel scaffold; baseline (speedup 1.0000x reference)
import jax
import jax.numpy as jnp
from jax.experimental import pallas as pl


def kernel(x, thresholds, ordinals, weights):
    raise NotImplementedError("write your pallas kernel here")


# speedup vs baseline: 417.4434x; 417.4434x over previous
import jax
import jax.numpy as jnp
from jax import lax
from jax.experimental import pallas as pl
from jax.experimental.pallas import tpu as pltpu
import jax.experimental.pallas.tpu_sc as plsc

L = 16
NC = 2
NS = 16
NW = NC * NS


def _forest_body(x_hbm, thr_hbm, ord_hbm, w_hbm, out_hbm,
                 x_v, thr_v, ord_v, w_v, out_v):
    B, C = x_hbm.shape
    T, NI = thr_hbm.shape
    depth = NI.bit_length()
    ROWS = x_v.shape[0]
    TB = thr_v.shape[0]
    n_chunks = B // (NW * ROWS)
    n_tb = T // TB
    n_tg = TB // L

    wid = lax.axis_index("s") * NC + lax.axis_index("c")
    row0_w = wid * (B // NW)
    iota = lax.iota(jnp.int32, L)

    def do_chunk(c, carry):
        row0 = row0_w + c * ROWS
        pltpu.sync_copy(x_hbm.at[pl.ds(row0, ROWS)], x_v)

        def do_tb(tb, carry2):
            t0 = tb * TB
            pltpu.sync_copy(thr_hbm.at[pl.ds(t0, TB)], thr_v)
            pltpu.sync_copy(ord_hbm.at[pl.ds(t0, TB)], ord_v)
            pltpu.sync_copy(w_hbm.at[pl.ds(t0, TB)], w_v)

            def do_row(r, carry3):
                rvec = jnp.full((L,), r, jnp.int32)
                for tg in range(n_tg):
                    tvec = tg * L + iota
                    node = jnp.zeros((L,), jnp.int32)
                    minm = None
                    for _ in range(depth):
                        o = plsc.load_gather(ord_v, [tvec, node])
                        th = plsc.load_gather(thr_v, [tvec, node])
                        feat = plsc.load_gather(x_v, [rvec, o])
                        m = feat - th
                        a = jnp.abs(m)
                        minm = a if minm is None else jnp.minimum(minm, a)
                        node = 2 * node + 1 + (m > 0).astype(jnp.int32)
                    leaf = node - NI
                    w = plsc.load_gather(w_v, [tvec, leaf])
                    out_v[r, pl.ds(tb * TB + tg * L, L)] = minm * w
                return carry3

            lax.fori_loop(0, ROWS, do_row, 0)
            return carry2

        lax.fori_loop(0, n_tb, do_tb, 0)
        pltpu.sync_copy(out_v, out_hbm.at[pl.ds(row0, ROWS)])
        return carry

    lax.fori_loop(0, n_chunks, do_chunk, 0)


def kernel(x, thresholds, ordinals, weights):
    B, C = x.shape
    T, NI = thresholds.shape
    ROWS = min(32, B // NW)
    TB = min(64, T)
    mesh = plsc.VectorSubcoreMesh(
        core_axis_name="c", subcore_axis_name="s",
        num_cores=NC, num_subcores=NS)
    f = pl.kernel(
        _forest_body,
        out_type=jax.ShapeDtypeStruct((B, T), jnp.float32),
        mesh=mesh,
        compiler_params=pltpu.CompilerParams(needs_layout_passes=False),
        scratch_types=[
            pltpu.VMEM((ROWS, C), jnp.float32),
            pltpu.VMEM((TB, NI), jnp.float32),
            pltpu.VMEM((TB, NI), jnp.int32),
            pltpu.VMEM((TB, NI + 1), jnp.float32),
            pltpu.VMEM((ROWS, T), jnp.float32),
        ],
    )
    return f(x, thresholds, ordinals, weights)


# speedup vs baseline: 1061.5869x; 2.5431x over previous
import jax
import jax.numpy as jnp
from jax import lax
from jax.experimental import pallas as pl
from jax.experimental.pallas import tpu as pltpu
import jax.experimental.pallas.tpu_sc as plsc

L = 16
NC = 2
NS = 16
NW = NC * NS
RU = 2


def _forest_body(x_hbm, thr_hbm, ord_hbm, w_hbm, out_hbm,
                 x_v, thr_v, ord_v, w_v, out_v):
    B, T = out_hbm.shape
    C = x_hbm.shape[0] // B
    NL = thr_hbm.shape[0] // T
    NI = NL - 1
    depth = NI.bit_length()
    ROWS = out_v.shape[0]
    TB = thr_v.shape[0] // NL
    n_chunks = B // (NW * ROWS)
    n_tb = T // TB
    n_tg = TB // L

    wid = lax.axis_index("s") * NC + lax.axis_index("c")
    row0_w = wid * (B // NW)
    iota = lax.iota(jnp.int32, L)
    tbases = [(tg * L + iota) * NL for tg in range(n_tg)]
    wbases = [tb - NI for tb in tbases]

    def do_chunk(c, carry):
        row0 = row0_w + c * ROWS
        pltpu.sync_copy(x_hbm.at[pl.ds(row0 * C, ROWS * C)], x_v)

        def do_tb(tb, carry2):
            t0 = tb * TB
            col0 = t0
            pltpu.sync_copy(thr_hbm.at[pl.ds(t0 * NL, TB * NL)], thr_v)
            pltpu.sync_copy(ord_hbm.at[pl.ds(t0 * NL, TB * NL)], ord_v)
            pltpu.sync_copy(w_hbm.at[pl.ds(t0 * NL, TB * NL)], w_v)

            def do_rows(r2, carry3):
                r = r2 * RU
                nch = RU * n_tg
                rowbase = [jnp.full((L,), (r + dr) * C, jnp.int32)
                           for dr in range(RU)]
                node = [None] * nch
                minm = [None] * nch
                mcur = [None] * nch
                ocur = [None] * nch
                thcur = [None] * nch
                fcur = [None] * nch
                idx = [None] * nch
                for lvl in range(depth):
                    for k in range(nch):
                        tg = k % n_tg
                        if lvl == 0:
                            idx[k] = tbases[tg]
                        else:
                            idx[k] = tbases[tg] + node[k]
                    for k in range(nch):
                        ocur[k] = plsc.load_gather(ord_v, [idx[k]])
                    for k in range(nch):
                        thcur[k] = plsc.load_gather(thr_v, [idx[k]])
                    for k in range(nch):
                        dr = k // n_tg
                        fcur[k] = plsc.load_gather(x_v, [rowbase[dr] + ocur[k]])
                    for k in range(nch):
                        mcur[k] = fcur[k] - thcur[k]
                    for k in range(nch):
                        a = jnp.abs(mcur[k])
                        minm[k] = a if lvl == 0 else jnp.minimum(minm[k], a)
                    for k in range(nch):
                        inc = jnp.where(mcur[k] > 0, 2, 1).astype(jnp.int32)
                        if lvl == 0:
                            node[k] = inc
                        else:
                            node[k] = node[k] + node[k] + inc
                for k in range(nch):
                    tg = k % n_tg
                    w = plsc.load_gather(w_v, [wbases[tg] + node[k]])
                    dr = k // n_tg
                    out_v[r + dr, pl.ds(col0 + tg * L, L)] = minm[k] * w
                return carry3

            lax.fori_loop(0, ROWS // RU, do_rows, 0)
            return carry2

        lax.fori_loop(0, n_tb, do_tb, 0)
        pltpu.sync_copy(out_v, out_hbm.at[pl.ds(row0, ROWS)])
        return carry

    lax.fori_loop(0, n_chunks, do_chunk, 0)


def kernel(x, thresholds, ordinals, weights):
    B, C = x.shape
    T, NI = thresholds.shape
    NL = NI + 1
    ROWS = min(32, B // NW)
    TB = min(64, T)
    thr_p = jnp.pad(thresholds, ((0, 0), (0, 1))).reshape(-1)
    ord_p = jnp.pad(ordinals, ((0, 0), (0, 1))).reshape(-1)
    w_p = weights.reshape(-1)
    x_f = x.reshape(-1)
    mesh = plsc.VectorSubcoreMesh(
        core_axis_name="c", subcore_axis_name="s",
        num_cores=NC, num_subcores=NS)
    f = pl.kernel(
        _forest_body,
        out_type=jax.ShapeDtypeStruct((B, T), jnp.float32),
        mesh=mesh,
        compiler_params=pltpu.CompilerParams(needs_layout_passes=False),
        scratch_types=[
            pltpu.VMEM((ROWS * C,), jnp.float32),
            pltpu.VMEM((TB * NL,), jnp.float32),
            pltpu.VMEM((TB * NL,), jnp.int32),
            pltpu.VMEM((TB * NL,), jnp.float32),
            pltpu.VMEM((ROWS, T), jnp.float32),
        ],
    )
    return f(x_f, thr_p, ord_p, w_p)


# speedup vs baseline: 1103.8480x; 1.0398x over previous
import jax
import jax.numpy as jnp
from jax import lax
from jax.experimental import pallas as pl
from jax.experimental.pallas import tpu as pltpu
import jax.experimental.pallas.tpu_sc as plsc

L = 16
NC = 2
NS = 16
NW = NC * NS
RU = 4


def _forest_body(x_hbm, thr_hbm, ord_hbm, w_hbm, out_hbm,
                 x_v, thr_v, ord_v, w_v, out_v):
    B, T = out_hbm.shape
    C = x_hbm.shape[0] // B
    NL = thr_hbm.shape[0] // T
    NI = NL - 1
    depth = NI.bit_length()
    ROWS = out_v.shape[0]
    TB = thr_v.shape[0] // NL
    n_chunks = B // (NW * ROWS)
    n_tb = T // TB
    n_tg = TB // L

    wid = lax.axis_index("s") * NC + lax.axis_index("c")
    row0_w = wid * (B // NW)
    iota = lax.iota(jnp.int32, L)
    tbases = [(tg * L + iota) * NL for tg in range(n_tg)]
    wbases = [tb - NI for tb in tbases]

    def do_chunk(c, carry):
        row0 = row0_w + c * ROWS
        pltpu.sync_copy(x_hbm.at[pl.ds(row0 * C, ROWS * C)], x_v)

        def do_tb(tb, carry2):
            t0 = tb * TB
            col0 = t0
            pltpu.sync_copy(thr_hbm.at[pl.ds(t0 * NL, TB * NL)], thr_v)
            pltpu.sync_copy(ord_hbm.at[pl.ds(t0 * NL, TB * NL)], ord_v)
            pltpu.sync_copy(w_hbm.at[pl.ds(t0 * NL, TB * NL)], w_v)

            def do_rows(r2, carry3):
                r = r2 * RU
                nch = RU * n_tg
                rowbase = [jnp.full((L,), (r + dr) * C, jnp.int32)
                           for dr in range(RU)]
                node = [None] * nch
                minm = [None] * nch
                mcur = [None] * nch
                ocur = [None] * nch
                thcur = [None] * nch
                fcur = [None] * nch
                idx = [None] * nch
                for lvl in range(depth):
                    for k in range(nch):
                        tg = k % n_tg
                        if lvl == 0:
                            idx[k] = tbases[tg]
                        else:
                            idx[k] = tbases[tg] + node[k]
                    for k in range(nch):
                        ocur[k] = plsc.load_gather(ord_v, [idx[k]])
                    for k in range(nch):
                        thcur[k] = plsc.load_gather(thr_v, [idx[k]])
                    for k in range(nch):
                        dr = k // n_tg
                        fcur[k] = plsc.load_gather(x_v, [rowbase[dr] + ocur[k]])
                    for k in range(nch):
                        mcur[k] = fcur[k] - thcur[k]
                    for k in range(nch):
                        a = jnp.abs(mcur[k])
                        minm[k] = a if lvl == 0 else jnp.minimum(minm[k], a)
                    for k in range(nch):
                        inc = jnp.where(mcur[k] > 0, 2, 1).astype(jnp.int32)
                        if lvl == 0:
                            node[k] = inc
                        else:
                            node[k] = node[k] + node[k] + inc
                for k in range(nch):
                    tg = k % n_tg
                    w = plsc.load_gather(w_v, [wbases[tg] + node[k]])
                    dr = k // n_tg
                    out_v[r + dr, pl.ds(col0 + tg * L, L)] = minm[k] * w
                return carry3

            lax.fori_loop(0, ROWS // RU, do_rows, 0)
            return carry2

        lax.fori_loop(0, n_tb, do_tb, 0)
        pltpu.sync_copy(out_v, out_hbm.at[pl.ds(row0, ROWS)])
        return carry

    lax.fori_loop(0, n_chunks, do_chunk, 0)


def kernel(x, thresholds, ordinals, weights):
    B, C = x.shape
    T, NI = thresholds.shape
    NL = NI + 1
    ROWS = min(32, B // NW)
    TB = min(64, T)
    thr_p = jnp.pad(thresholds, ((0, 0), (0, 1))).reshape(-1)
    ord_p = jnp.pad(ordinals, ((0, 0), (0, 1))).reshape(-1)
    w_p = weights.reshape(-1)
    x_f = x.reshape(-1)
    mesh = plsc.VectorSubcoreMesh(
        core_axis_name="c", subcore_axis_name="s",
        num_cores=NC, num_subcores=NS)
    f = pl.kernel(
        _forest_body,
        out_type=jax.ShapeDtypeStruct((B, T), jnp.float32),
        mesh=mesh,
        compiler_params=pltpu.CompilerParams(needs_layout_passes=False),
        scratch_types=[
            pltpu.VMEM((ROWS * C,), jnp.float32),
            pltpu.VMEM((TB * NL,), jnp.float32),
            pltpu.VMEM((TB * NL,), jnp.int32),
            pltpu.VMEM((TB * NL,), jnp.float32),
            pltpu.VMEM((ROWS, T), jnp.float32),
        ],
    )
    return f(x_f, thr_p, ord_p, w_p)


# speedup vs baseline: 1111.9717x; 1.0074x over previous
import jax
import jax.numpy as jnp
from jax import lax
from jax.experimental import pallas as pl
from jax.experimental.pallas import tpu as pltpu
import jax.experimental.pallas.tpu_sc as plsc

L = 16
NC = 2
NS = 16
NW = NC * NS
RU = 2


def _forest_body(x_hbm, thr_hbm, ord_hbm, w_hbm, out_hbm,
                 x_v, thr_v, ord_v, w_v, out_v):
    B, T = out_hbm.shape
    C = x_hbm.shape[0] // B
    NL = thr_hbm.shape[0] // T
    NI = NL - 1
    depth = NI.bit_length()
    ROWS = out_v.shape[0]
    TB = thr_v.shape[0] // NL
    n_chunks = B // (NW * ROWS)
    n_tb = T // TB
    n_tg = TB // L

    wid = lax.axis_index("s") * NC + lax.axis_index("c")
    row0_w = wid * (B // NW)
    iota = lax.iota(jnp.int32, L)
    tbases = [(tg * L + iota) * NL for tg in range(n_tg)]
    wbases = [tb - NI for tb in tbases]

    def do_chunk(c, carry):
        row0 = row0_w + c * ROWS
        pltpu.sync_copy(x_hbm.at[pl.ds(row0 * C, ROWS * C)], x_v)

        def do_tb(tb, carry2):
            t0 = tb * TB
            col0 = t0
            pltpu.sync_copy(thr_hbm.at[pl.ds(t0 * NL, TB * NL)], thr_v)
            pltpu.sync_copy(ord_hbm.at[pl.ds(t0 * NL, TB * NL)], ord_v)
            pltpu.sync_copy(w_hbm.at[pl.ds(t0 * NL, TB * NL)], w_v)

            def do_rows(r2):
                r = r2 * RU
                nch = RU * n_tg
                rowbase = [jnp.full((L,), (r + dr) * C, jnp.int32)
                           for dr in range(RU)]
                node = [None] * nch
                minm = [None] * nch
                mcur = [None] * nch
                ocur = [None] * nch
                thcur = [None] * nch
                fcur = [None] * nch
                idx = [None] * nch
                for lvl in range(depth):
                    for k in range(nch):
                        tg = k % n_tg
                        if lvl == 0:
                            idx[k] = tbases[tg]
                        else:
                            idx[k] = tbases[tg] + node[k]
                    for k in range(nch):
                        ocur[k] = plsc.load_gather(ord_v, [idx[k]])
                    for k in range(nch):
                        thcur[k] = plsc.load_gather(thr_v, [idx[k]])
                    for k in range(nch):
                        dr = k // n_tg
                        fcur[k] = plsc.load_gather(x_v, [rowbase[dr] + ocur[k]])
                    for k in range(nch):
                        mcur[k] = fcur[k] - thcur[k]
                    for k in range(nch):
                        a = jnp.abs(mcur[k])
                        minm[k] = a if lvl == 0 else jnp.minimum(minm[k], a)
                    for k in range(nch):
                        inc = jnp.where(mcur[k] > 0, 2, 1).astype(jnp.int32)
                        if lvl == 0:
                            node[k] = inc
                        else:
                            node[k] = node[k] + node[k] + inc
                for k in range(nch):
                    tg = k % n_tg
                    w = plsc.load_gather(w_v, [wbases[tg] + node[k]])
                    dr = k // n_tg
                    out_v[r + dr, pl.ds(col0 + tg * L, L)] = minm[k] * w

            plsc.parallel_loop(0, ROWS // RU, 1, unroll=2)(do_rows)
            return carry2

        lax.fori_loop(0, n_tb, do_tb, 0)
        pltpu.sync_copy(out_v, out_hbm.at[pl.ds(row0, ROWS)])
        return carry

    lax.fori_loop(0, n_chunks, do_chunk, 0)


def kernel(x, thresholds, ordinals, weights):
    B, C = x.shape
    T, NI = thresholds.shape
    NL = NI + 1
    ROWS = min(32, B // NW)
    TB = min(64, T)
    thr_p = jnp.pad(thresholds, ((0, 0), (0, 1))).reshape(-1)
    ord_p = jnp.pad(ordinals, ((0, 0), (0, 1))).reshape(-1)
    w_p = weights.reshape(-1)
    x_f = x.reshape(-1)
    mesh = plsc.VectorSubcoreMesh(
        core_axis_name="c", subcore_axis_name="s",
        num_cores=NC, num_subcores=NS)
    f = pl.kernel(
        _forest_body,
        out_type=jax.ShapeDtypeStruct((B, T), jnp.float32),
        mesh=mesh,
        compiler_params=pltpu.CompilerParams(needs_layout_passes=False),
        scratch_types=[
            pltpu.VMEM((ROWS * C,), jnp.float32),
            pltpu.VMEM((TB * NL,), jnp.float32),
            pltpu.VMEM((TB * NL,), jnp.int32),
            pltpu.VMEM((TB * NL,), jnp.float32),
            pltpu.VMEM((ROWS, T), jnp.float32),
        ],
    )
    return f(x_f, thr_p, ord_p, w_p)
